# Initial kernel scaffold; baseline (speedup 1.0000x reference)
import jax
import jax.numpy as jnp
from jax.experimental import pallas as pl


def kernel(x, edge_index, weight_in, Wa_src, Wa_dst, Wv, Wm, Wg, Wo, weight_out):
    raise NotImplementedError("write your pallas kernel here")


# speedup vs baseline: 1.1507x; 1.1507x over previous
import functools

import jax
import jax.numpy as jnp
from jax.experimental import pallas as pl
from jax.experimental.pallas import tpu as pltpu

N = 10000
E = 320000
F_IN = 128
HID = 128
NCLS = 40
H = 8
DA = 16
DV = 16
DM = 64
LAYERS = 2
NEG = 0.1

NPAD = 10240
RB = 512


def _mm_kernel(x_ref, w_ref, o_ref):
    o_ref[...] = jnp.dot(x_ref[...], w_ref[...],
                         preferred_element_type=jnp.float32)


def _matmul(x, w):
    m, k = x.shape
    k2, n = w.shape
    grid = (m // RB,)
    return pl.pallas_call(
        _mm_kernel,
        grid=grid,
        in_specs=[pl.BlockSpec((RB, k), lambda i: (i, 0)),
                  pl.BlockSpec((k2, n), lambda i: (0, 0))],
        out_specs=pl.BlockSpec((RB, n), lambda i: (i, 0)),
        out_shape=jax.ShapeDtypeStruct((m, n), jnp.float32),
    )(x, w)


def _gate_out_kernel(h_ref, nmax_ref, nmean_ref, aggp_ref, den_ref,
                     wg_ref, wo_ref, o_ref):
    h = h_ref[...]
    gcat = jnp.concatenate([h, nmax_ref[...], nmean_ref[...]], axis=1)
    g = jax.nn.sigmoid(jnp.dot(gcat, wg_ref[...],
                               preferred_element_type=jnp.float32))
    agg = aggp_ref[...] / den_ref[...]
    gated = jnp.repeat(g, DV, axis=1) * agg
    cat = jnp.concatenate([h, gated], axis=1)
    o = jnp.dot(cat, wo_ref[...], preferred_element_type=jnp.float32)
    o_ref[...] = jnp.where(o >= 0, o, NEG * o)


def _gate_out(h, nmax, nmean, aggp, den, wg, wo):
    m = h.shape[0]
    grid = (m // RB,)
    rb = lambda c: pl.BlockSpec((RB, c), lambda i: (i, 0))
    full = lambda a, b: pl.BlockSpec((a, b), lambda i: (0, 0))
    return pl.pallas_call(
        _gate_out_kernel,
        grid=grid,
        in_specs=[rb(HID), rb(DM), rb(HID), rb(H * DV), rb(H * DV),
                  full(HID + DM + HID, H), full(HID + H * DV, HID)],
        out_specs=rb(HID),
        out_shape=jax.ShapeDtypeStruct((m, HID), jnp.float32),
    )(h, nmax, nmean, aggp, den, wg, wo)


def _final_kernel(h_ref, w_ref, o_ref):
    o = jnp.dot(h_ref[...], w_ref[...], preferred_element_type=jnp.float32)
    m = jnp.max(o, axis=1, keepdims=True)
    lse = jnp.log(jnp.sum(jnp.exp(o - m), axis=1, keepdims=True)) + m
    o_ref[...] = o - lse


def _final(h, w):
    m = h.shape[0]
    grid = (m // RB,)
    return pl.pallas_call(
        _final_kernel,
        grid=grid,
        in_specs=[pl.BlockSpec((RB, HID), lambda i: (i, 0)),
                  pl.BlockSpec((HID, NCLS), lambda i: (0, 0))],
        out_specs=pl.BlockSpec((RB, NCLS), lambda i: (i, 0)),
        out_shape=jax.ShapeDtypeStruct((m, NCLS), jnp.float32),
    )(h, w)


def kernel(x, edge_index, weight_in, Wa_src, Wa_dst, Wv, Wm, Wg, Wo, weight_out):
    src = edge_index[0]
    dst = edge_index[1]
    xp = jnp.pad(x, ((0, NPAD - N), (0, 0)))
    h = _matmul(xp, weight_in)
    for i in range(LAYERS):
        wcat = jnp.concatenate([Wa_dst[i], Wa_src[i], Wv[i], Wm[i]], axis=1)
        proj = _matmul(h, wcat)
        q = proj[:N, 0:HID]
        k = proj[:N, HID:2 * HID]
        v = proj[:N, 2 * HID:3 * HID]
        mz = proj[:N, 3 * HID:3 * HID + DM]
        hn = h[:N]
        logits = jnp.sum((q[dst].reshape(E, H, DA)) * (k[src].reshape(E, H, DA)),
                         axis=-1)
        p = jnp.exp(logits)
        den = jax.ops.segment_sum(p, dst, num_segments=N)
        aggp = jax.ops.segment_sum(p[:, :, None] * v[src].reshape(E, H, DV),
                                   dst, num_segments=N).reshape(N, H * DV)
        nmax = jax.ops.segment_max(mz[src], dst, num_segments=N)
        nmax = jnp.where(jnp.isfinite(nmax), nmax, 0.0)
        deg = jax.ops.segment_sum(jnp.ones((E,), jnp.float32), dst,
                                  num_segments=N)
        nsum = jax.ops.segment_sum(hn[src], dst, num_segments=N)
        nmean = nsum / jnp.maximum(deg, 1.0)[:, None]
        denf = jnp.maximum(den, 1e-30)
        denr = jnp.repeat(denf, DV, axis=1)
        pad = lambda a: jnp.pad(a, ((0, NPAD - N), (0, 0)))
        h = _gate_out(pad(hn), pad(nmax), pad(nmean), pad(aggp),
                      jnp.pad(denr, ((0, NPAD - N), (0, 0)),
                              constant_values=1.0),
                      Wg[i], Wo[i])
    out = _final(h, weight_out)
    return out[:N]


# speedup vs baseline: 2.0707x; 1.7996x over previous
import functools

import jax
import jax.numpy as jnp
from jax import lax
from jax.experimental import pallas as pl
from jax.experimental.pallas import tpu as pltpu
from jax.experimental.pallas import tpu_sc as plsc

N = 10000
E = 320000
F_IN = 128
HID = 128
NCLS = 40
H = 8
DA = 16
DV = 16
DM = 64
LAYERS = 2
NEG = 0.1

NPAD = 10240
RB = 512

NC = 2
NS = 16
NW = NC * NS
NSWEEP = 2
RS = 160
R = NSWEEP * RS
NOUT = NW * R
CHUNK = 1280
NVREG = CHUNK // 16
NCHUNK = E // CHUNK

CB = 32
CBP = CB + 16
KW = 512

_mesh = plsc.VectorSubcoreMesh(core_axis_name="c", subcore_axis_name="s",
                               num_cores=NC, num_subcores=NS)


def _zero_i32(ref, n):
    z = jnp.zeros((16,), jnp.int32)
    for i in range(n // 16):
        ref[pl.ds(i * 16, 16)] = z


@functools.partial(
    pl.kernel,
    out_type=(jax.ShapeDtypeStruct((NOUT, H * DV), jnp.float32),
              jax.ShapeDtypeStruct((NOUT, 16), jnp.float32),
              jax.ShapeDtypeStruct((NOUT, HID), jnp.float32),
              jax.ShapeDtypeStruct((NOUT, DM), jnp.float32)),
    mesh=_mesh,
    compiler_params=pltpu.CompilerParams(needs_layout_passes=False),
    scratch_types=[
        pltpu.VMEM((RS, HID), jnp.float32),
        pltpu.VMEM((RS, H * DV), jnp.float32),
        pltpu.VMEM((RS, 16), jnp.float32),
        pltpu.VMEM((RS, HID), jnp.float32),
        pltpu.VMEM((RS, DM), jnp.float32),
        pltpu.VMEM((CHUNK,), jnp.int32),
        pltpu.VMEM((CHUNK,), jnp.int32),
        pltpu.VMEM((CBP,), jnp.int32),
        pltpu.VMEM((CBP,), jnp.int32),
        pltpu.VMEM((CBP, KW), jnp.float32),
        pltpu.SMEM((RS,), jnp.float32),
        pltpu.SemaphoreType.DMA,
        pltpu.SemaphoreType.DMA,
    ],
)
def _sc_edge(dst_hbm, src_hbm, q_hbm, tab_hbm,
             agg_hbm, den_hbm, nmean_hbm, nmax_hbm,
             q_own, acc_agg, acc_den, acc_sum, acc_max,
             dstbuf, srcbuf, comp_src, comp_dl, rows, deg, csem, gsem):
    wid = lax.axis_index("s") * NC + lax.axis_index("c")
    fz = jnp.zeros((16,), jnp.float32)
    neg = jnp.full((16,), -3.0e38, jnp.float32)
    iot = lax.broadcasted_iota(jnp.int32, (16,), 0)
    onehot = [(iot == hh).astype(jnp.float32) for hh in range(H)]

    for s in range(NSWEEP):
        lo = wid * R + s * RS
        hi = lo + RS

        pltpu.async_copy(q_hbm.at[pl.ds(lo, RS)], q_own, gsem).wait()
        _zero_i32(comp_src, CBP)

        def init_body(r, _):
            for kk in range(H):
                acc_agg[r, pl.ds(kk * 16, 16)] = fz
            acc_den[r, pl.ds(0, 16)] = fz
            for kk in range(HID // 16):
                acc_sum[r, pl.ds(kk * 16, 16)] = fz
            for kk in range(DM // 16):
                acc_max[r, pl.ds(kk * 16, 16)] = neg
            deg[r] = 0.0
            return 0

        lax.fori_loop(0, RS, init_body, 0)

        def flush(wp):
            pltpu.async_copy(tab_hbm.at[comp_src], rows, gsem).wait()

            def edge_body(j, _):
                dl = comp_dl[pl.ds(j, 16)][0]
                denv = fz
                for hh in range(H):
                    vq = q_own[dl, pl.ds(hh * 16, 16)]
                    vk = rows[j, pl.ds(hh * 16, 16)]
                    l = jnp.sum(vq * vk)
                    pv = jnp.exp(l + fz)
                    vv = rows[j, pl.ds(HID + hh * 16, 16)]
                    plsc.addupdate(acc_agg.at[dl, pl.ds(hh * 16, 16)], pv * vv)
                    denv = denv + pv * onehot[hh]
                plsc.addupdate(acc_den.at[dl, pl.ds(0, 16)], denv)
                for kk in range(HID // 16):
                    plsc.addupdate(acc_sum.at[dl, pl.ds(kk * 16, 16)],
                                   rows[j, pl.ds(2 * HID + kk * 16, 16)])
                for kk in range(DM // 16):
                    cur = acc_max[dl, pl.ds(kk * 16, 16)]
                    acc_max[dl, pl.ds(kk * 16, 16)] = jnp.maximum(
                        cur, rows[j, pl.ds(3 * HID + kk * 16, 16)])
                deg[dl] = deg[dl] + 1.0
                return 0

            lax.fori_loop(0, wp, edge_body, 0)

        def chunk_body(c, wp):
            pltpu.async_copy(dst_hbm.at[pl.ds(c * CHUNK, CHUNK)], dstbuf,
                             csem).wait()
            pltpu.async_copy(src_hbm.at[pl.ds(c * CHUNK, CHUNK)], srcbuf,
                             csem).wait()

            def vec_body(i, wp):
                vdst = dstbuf[pl.ds(i * 16, 16)]
                vsrc = srcbuf[pl.ds(i * 16, 16)]
                m = (vdst >= lo) & (vdst < hi)
                cnt = jnp.max(plsc.all_reduce_population_count(m))
                plsc.store_compressed(comp_src.at[pl.ds(wp, 16)], vsrc,
                                      mask=m)
                plsc.store_compressed(comp_dl.at[pl.ds(wp, 16)], vdst - lo,
                                      mask=m)
                wp2 = wp + cnt
                do = wp2 > CB - 16

                @pl.when(do)
                def _():
                    flush(wp2)

                return jnp.where(do, jnp.int32(0), wp2)

            return lax.fori_loop(0, NVREG, vec_body, wp)

        wp = lax.fori_loop(0, NCHUNK, chunk_body, jnp.int32(0))
        flush(wp)

        def fin_body(r, _):
            dv = fz + deg[r]
            rec = 1.0 / jnp.maximum(dv, 1.0)
            sel = jnp.where(dv > 0.0, 1.0, 0.0)
            for kk in range(HID // 16):
                acc_sum[r, pl.ds(kk * 16, 16)] = (
                    acc_sum[r, pl.ds(kk * 16, 16)] * rec)
            for kk in range(DM // 16):
                acc_max[r, pl.ds(kk * 16, 16)] = (
                    acc_max[r, pl.ds(kk * 16, 16)] * sel)
            return 0

        lax.fori_loop(0, RS, fin_body, 0)

        pltpu.async_copy(acc_agg, agg_hbm.at[pl.ds(lo, RS)], gsem).wait()
        pltpu.async_copy(acc_den, den_hbm.at[pl.ds(lo, RS)], gsem).wait()
        pltpu.async_copy(acc_sum, nmean_hbm.at[pl.ds(lo, RS)], gsem).wait()
        pltpu.async_copy(acc_max, nmax_hbm.at[pl.ds(lo, RS)], gsem).wait()


def _mm_kernel(x_ref, w_ref, o_ref):
    o_ref[...] = jnp.dot(x_ref[...], w_ref[...],
                         preferred_element_type=jnp.float32)


def _matmul(x, w):
    m, k = x.shape
    k2, n = w.shape
    return pl.pallas_call(
        _mm_kernel,
        grid=(m // RB,),
        in_specs=[pl.BlockSpec((RB, k), lambda i: (i, 0)),
                  pl.BlockSpec((k2, n), lambda i: (0, 0))],
        out_specs=pl.BlockSpec((RB, n), lambda i: (i, 0)),
        out_shape=jax.ShapeDtypeStruct((m, n), jnp.float32),
    )(x, w)


def _gate_out_kernel(h_ref, nmax_ref, nmean_ref, aggp_ref, den_ref,
                     wg_ref, wo_ref, o_ref):
    h = h_ref[...]
    gcat = jnp.concatenate([h, nmax_ref[...], nmean_ref[...]], axis=1)
    g = jax.nn.sigmoid(jnp.dot(gcat, wg_ref[...],
                               preferred_element_type=jnp.float32))
    agg = aggp_ref[...] / den_ref[...]
    gated = jnp.repeat(g, DV, axis=1) * agg
    cat = jnp.concatenate([h, gated], axis=1)
    o = jnp.dot(cat, wo_ref[...], preferred_element_type=jnp.float32)
    o_ref[...] = jnp.where(o >= 0, o, NEG * o)


def _gate_out(h, nmax, nmean, aggp, den, wg, wo):
    m = h.shape[0]
    rb = lambda c: pl.BlockSpec((RB, c), lambda i: (i, 0))
    full = lambda a, b: pl.BlockSpec((a, b), lambda i: (0, 0))
    return pl.pallas_call(
        _gate_out_kernel,
        grid=(m // RB,),
        in_specs=[rb(HID), rb(DM), rb(HID), rb(H * DV), rb(H * DV),
                  full(HID + DM + HID, H), full(HID + H * DV, HID)],
        out_specs=rb(HID),
        out_shape=jax.ShapeDtypeStruct((m, HID), jnp.float32),
    )(h, nmax, nmean, aggp, den, wg, wo)


def _final_kernel(h_ref, w_ref, o_ref):
    o = jnp.dot(h_ref[...], w_ref[...], preferred_element_type=jnp.float32)
    m = jnp.max(o, axis=1, keepdims=True)
    lse = jnp.log(jnp.sum(jnp.exp(o - m), axis=1, keepdims=True)) + m
    o_ref[...] = o - lse


def _final(h, w):
    m = h.shape[0]
    return pl.pallas_call(
        _final_kernel,
        grid=(m // RB,),
        in_specs=[pl.BlockSpec((RB, HID), lambda i: (i, 0)),
                  pl.BlockSpec((HID, NCLS), lambda i: (0, 0))],
        out_specs=pl.BlockSpec((RB, NCLS), lambda i: (i, 0)),
        out_shape=jax.ShapeDtypeStruct((m, NCLS), jnp.float32),
    )(h, w)


def kernel(x, edge_index, weight_in, Wa_src, Wa_dst, Wv, Wm, Wg, Wo, weight_out):
    src = edge_index[0]
    dst = edge_index[1]
    xp = jnp.pad(x, ((0, NPAD - N), (0, 0)))
    h = _matmul(xp, weight_in)
    for i in range(LAYERS):
        wcat = jnp.concatenate([Wa_dst[i], Wa_src[i], Wv[i], Wm[i]], axis=1)
        proj = _matmul(h, wcat)
        qT = proj[:, 0:HID]
        tab = jnp.concatenate(
            [proj[:N, HID:3 * HID],
             h[:N],
             proj[:N, 3 * HID:3 * HID + DM],
             jnp.zeros((N, KW - 3 * HID - DM), jnp.float32)], axis=1)
        aggp, den, nmean, nmax = _sc_edge(dst, src, qT, tab)
        den8 = jnp.maximum(den[:, :H], 1e-30)
        denr = jnp.repeat(den8, DV, axis=1)
        h = _gate_out(h, nmax, nmean, aggp, denr, Wg[i], Wo[i])
    out = _final(h, weight_out)
    return out[:N]
